# baseline (speedup 1.0000x reference)
import jax
import jax.numpy as jnp
from jax.experimental import pallas as pl
from jax.experimental.pallas import tpu as pltpu

NUM_EXPERTS = 8
IN_FEATURES = 1024
OUT_FEATURES = 1024
N_TOKENS = 8192
BLOCK_N = 512
HALF_E = NUM_EXPERTS // 2
HALF_K = HALF_E * IN_FEATURES


def _moe_body(x_ref, gw_ref, gb_ref, ew0_ref, ew1_ref, eb_ref, out_ref, xs0_ref, xs1_ref):
    x = x_ref[...]
    logits = (
        jnp.dot(x, gw_ref[...], preferred_element_type=jnp.float32) + gb_ref[...]
    )
    m = jnp.max(logits, axis=-1, keepdims=True)
    ex = jnp.exp(logits - m)
    s = ex / jnp.sum(ex, axis=-1, keepdims=True)
    for h, xs_ref in enumerate((xs0_ref, xs1_ref)):
        for j in range(HALF_E):
            e = h * HALF_E + j
            xs_ref[:, j * IN_FEATURES : (j + 1) * IN_FEATURES] = s[:, e : e + 1] * x
    out_ref[...] = (
        jnp.dot(xs0_ref[...], ew0_ref[...], preferred_element_type=jnp.float32)
        + jnp.dot(xs1_ref[...], ew1_ref[...], preferred_element_type=jnp.float32)
        + jnp.dot(s, eb_ref[...], preferred_element_type=jnp.float32)
    )


@jax.jit
def kernel(x, gate_W, gate_b, expert_W, expert_b):
    n_blocks = N_TOKENS // BLOCK_N
    ew = expert_W.reshape(NUM_EXPERTS * IN_FEATURES, OUT_FEATURES)
    out = pl.pallas_call(
        _moe_body,
        grid=(n_blocks,),
        in_specs=[
            pl.BlockSpec((BLOCK_N, IN_FEATURES), lambda i: (i, 0)),
            pl.BlockSpec((IN_FEATURES, NUM_EXPERTS), lambda i: (0, 0)),
            pl.BlockSpec((1, NUM_EXPERTS), lambda i: (0, 0)),
            pl.BlockSpec((HALF_K, OUT_FEATURES), lambda i: (0, 0)),
            pl.BlockSpec((HALF_K, OUT_FEATURES), lambda i: (0, 0)),
            pl.BlockSpec((NUM_EXPERTS, OUT_FEATURES), lambda i: (0, 0)),
        ],
        out_specs=pl.BlockSpec((BLOCK_N, OUT_FEATURES), lambda i: (i, 0)),
        out_shape=jax.ShapeDtypeStruct((N_TOKENS, OUT_FEATURES), jnp.float32),
        scratch_shapes=[
            pltpu.VMEM((BLOCK_N, HALF_K), jnp.float32),
            pltpu.VMEM((BLOCK_N, HALF_K), jnp.float32),
        ],
        compiler_params=pltpu.CompilerParams(
            dimension_semantics=("arbitrary",),
        ),
    )(x, gate_W, gate_b.reshape(1, NUM_EXPERTS), ew[:HALF_K], ew[HALF_K:], expert_b)
    return out


# speedup vs baseline: 1.1256x; 1.1256x over previous
import jax
import jax.numpy as jnp
from jax.experimental import pallas as pl
from jax.experimental.pallas import tpu as pltpu

NUM_EXPERTS = 8
IN_FEATURES = 1024
OUT_FEATURES = 1024
N_TOKENS = 8192
BLOCK_N = 512


def _moe_body(x_ref, gw_ref, gb_ref, ew_ref, eb_ref, out_ref, xs_ref):
    x = x_ref[...]
    logits = (
        jnp.dot(x, gw_ref[...], preferred_element_type=jnp.float32) + gb_ref[...]
    )
    m = jnp.max(logits, axis=-1, keepdims=True)
    ex = jnp.exp(logits - m)
    s = ex / jnp.sum(ex, axis=-1, keepdims=True)
    for e in range(NUM_EXPERTS):
        xs_ref[:, e * IN_FEATURES : (e + 1) * IN_FEATURES] = s[:, e : e + 1] * x
    out_ref[...] = jnp.dot(
        xs_ref[...], ew_ref[...], preferred_element_type=jnp.float32
    ) + jnp.dot(s, eb_ref[...], preferred_element_type=jnp.float32)


@jax.jit
def kernel(x, gate_W, gate_b, expert_W, expert_b):
    n_blocks = N_TOKENS // BLOCK_N
    ew = expert_W.reshape(NUM_EXPERTS * IN_FEATURES, OUT_FEATURES)
    out = pl.pallas_call(
        _moe_body,
        grid=(n_blocks,),
        in_specs=[
            pl.BlockSpec((BLOCK_N, IN_FEATURES), lambda i: (i, 0)),
            pl.BlockSpec((IN_FEATURES, NUM_EXPERTS), lambda i: (0, 0)),
            pl.BlockSpec((1, NUM_EXPERTS), lambda i: (0, 0)),
            pl.BlockSpec((NUM_EXPERTS * IN_FEATURES, OUT_FEATURES), lambda i: (0, 0)),
            pl.BlockSpec((NUM_EXPERTS, OUT_FEATURES), lambda i: (0, 0)),
        ],
        out_specs=pl.BlockSpec((BLOCK_N, OUT_FEATURES), lambda i: (i, 0)),
        out_shape=jax.ShapeDtypeStruct((N_TOKENS, OUT_FEATURES), jnp.float32),
        scratch_shapes=[
            pltpu.VMEM((BLOCK_N, NUM_EXPERTS * IN_FEATURES), jnp.float32)
        ],
        compiler_params=pltpu.CompilerParams(
            dimension_semantics=("arbitrary",),
        ),
    )(x, gate_W, gate_b.reshape(1, NUM_EXPERTS), ew, expert_b)
    return out
